# Initial kernel scaffold; baseline (speedup 1.0000x reference)
import jax
import jax.numpy as jnp
from jax.experimental import pallas as pl


def kernel(x, edge_index, edge_type, batch, params):
    raise NotImplementedError("write your pallas kernel here")


# speedup vs baseline: 6.3683x; 6.3683x over previous
import functools

import jax
import jax.numpy as jnp
from jax import lax
from jax.experimental import pallas as pl
from jax.experimental.pallas import tpu as pltpu
from jax.experimental.pallas import tpu_sc as plsc

N = 10000
E = 320000
D = 128
HD = 64
R = 3
NGRAPH = 16
NSUB = 16
SACC = 30208
HACC = 30720
DUMP = 30000
TPT = SACC // NSUB
EPAD = 321536
ECH = 128
CHUNKS = EPAD // NSUB // ECH
ECNT = E // NSUB
CCH = 80
CNCH = ECNT // CCH
NPAD = 10240
BN = 512
PBN = 400


def _prep_body(dst_hbm, et_hbm, inv_hbm, dst3_hbm, dbuf, ebuf, obuf, onesb,
               sumbuf, spm):
    c = lax.axis_index("c")
    s = lax.axis_index("s")

    @pl.when(c == 1)
    def _dst3():
        def chunk(k, carry):
            base = s * ECNT + k * CCH
            pltpu.sync_copy(dst_hbm.at[pl.ds(base, CCH)], dbuf)
            pltpu.sync_copy(et_hbm.at[pl.ds(base, CCH)], ebuf)
            for j in range(CCH // 16):
                sl = pl.ds(j * 16, 16)
                obuf[sl] = dbuf[sl] * 3 + ebuf[sl]
            pltpu.sync_copy(obuf, dst3_hbm.at[pl.ds(base, CCH)])
            return carry
        lax.fori_loop(0, CNCH, chunk, 0)

        @pl.when(s == 0)
        def _tail():
            for j in range(4):
                obuf[pl.ds(j * 16, 16)] = jnp.full((16,), DUMP, jnp.int32)
            def tail(k, carry):
                pltpu.sync_copy(obuf.at[pl.ds(0, 64)],
                                dst3_hbm.at[pl.ds(E + k * 64, 64)])
                return carry
            lax.fori_loop(0, (EPAD - E) // 64, tail, 0)

    @pl.when(c == 0)
    def _counts():
        zz = jnp.zeros((16,), jnp.float32)
        def zero(i, carry):
            sumbuf[pl.ds(i * 16, 16)] = zz
            return carry
        lax.fori_loop(0, TPT // 16, zero, 0)
        pltpu.sync_copy(sumbuf, spm.at[pl.ds(s * TPT, TPT)])
        ones = jnp.ones((16,), jnp.float32)
        for j in range(CCH // 16):
            onesb[pl.ds(j * 16, 16)] = ones
        plsc.subcore_barrier()
        def chunk(k, carry):
            base = s * ECNT + k * CCH
            pltpu.sync_copy(dst_hbm.at[pl.ds(base, CCH)], dbuf)
            pltpu.sync_copy(et_hbm.at[pl.ds(base, CCH)], ebuf)
            for j in range(CCH // 16):
                sl = pl.ds(j * 16, 16)
                obuf[sl] = dbuf[sl] * 3 + ebuf[sl]
            pltpu.sync_copy(onesb, spm.at[obuf], add=True)
            return carry
        lax.fori_loop(0, CNCH, chunk, 0)
        plsc.subcore_barrier()
        pltpu.sync_copy(spm.at[pl.ds(s * TPT, TPT)], sumbuf)
        def red(j, carry):
            sl = pl.ds(j * 16, 16)
            sumbuf[sl] = 1.0 / jnp.maximum(sumbuf[sl], 1.0)
            return carry
        lax.fori_loop(0, TPT // 16, red, 0)
        pltpu.sync_copy(sumbuf, inv_hbm.at[pl.ds(s * TPT, TPT)])


@functools.lru_cache(maxsize=None)
def _prep_kernel():
    mesh = plsc.VectorSubcoreMesh(core_axis_name="c", subcore_axis_name="s")
    return pl.kernel(
        _prep_body,
        out_type=(jax.ShapeDtypeStruct((HACC,), jnp.float32),
                  jax.ShapeDtypeStruct((EPAD,), jnp.int32)),
        mesh=mesh,
        scratch_types=[
            pltpu.VMEM((CCH,), jnp.int32),
            pltpu.VMEM((CCH,), jnp.int32),
            pltpu.VMEM((CCH,), jnp.int32),
            pltpu.VMEM((CCH,), jnp.float32),
            pltpu.VMEM((TPT,), jnp.float32),
            pltpu.VMEM_SHARED((SACC,), jnp.float32),
        ],
        compiler_params=pltpu.CompilerParams(use_tc_tiling_on_sc=False),
    )


def _agg_body(src_hbm, dst3_hbm, h2_hbm, a0_hbm, a1_hbm, srcv, rowv, rows,
              acc, sem):
    c = lax.axis_index("c")
    s = lax.axis_index("s")
    zz = jnp.zeros((16,), jnp.float32)
    def zrows(i, carry):
        for j in range(HD // 16):
            rows[i, pl.ds(j * 16, 16)] = zz
        return carry
    lax.fori_loop(0, ECH, zrows, 0)
    for m in range(TPT // ECH):
        pltpu.sync_copy(rows, acc.at[pl.ds(s * TPT + m * ECH, ECH)])
    pltpu.sync_copy(rows, acc.at[pl.ds(s * TPT + TPT - ECH, ECH)])
    plsc.subcore_barrier()

    def chunk(k, carry):
        base = s * (CHUNKS * ECH) + k * ECH
        pltpu.sync_copy(src_hbm.at[pl.ds(base, ECH)], srcv)
        pltpu.sync_copy(dst3_hbm.at[pl.ds(base, ECH)], rowv)
        for j in range(ECH // 16):
            sl = pl.ds(j * 16, 16)
            srcv[sl] = srcv[sl] * 2 + c
        pltpu.async_copy(h2_hbm.at[srcv], rows, sem).wait()
        pltpu.sync_copy(rows, acc.at[rowv], add=True)
        return carry
    lax.fori_loop(0, CHUNKS, chunk, 0)
    plsc.subcore_barrier()

    @pl.when(c == 0)
    def _out0():
        pltpu.sync_copy(acc.at[pl.ds(s * TPT, TPT)],
                        a0_hbm.at[pl.ds(s * TPT, TPT)])

    @pl.when(c == 1)
    def _out1():
        pltpu.sync_copy(acc.at[pl.ds(s * TPT, TPT)],
                        a1_hbm.at[pl.ds(s * TPT, TPT)])


@functools.lru_cache(maxsize=None)
def _agg_kernel():
    mesh = plsc.VectorSubcoreMesh(core_axis_name="c", subcore_axis_name="s")
    return pl.kernel(
        _agg_body,
        out_type=(jax.ShapeDtypeStruct((HACC, HD), jnp.float32),
                  jax.ShapeDtypeStruct((HACC, HD), jnp.float32)),
        mesh=mesh,
        scratch_types=[
            pltpu.VMEM((ECH,), jnp.int32),
            pltpu.VMEM((ECH,), jnp.int32),
            pltpu.VMEM((ECH, HD), jnp.float32),
            pltpu.VMEM_SHARED((SACC, HD), jnp.float32),
            pltpu.SemaphoreType.DMA,
        ],
        compiler_params=pltpu.CompilerParams(use_tc_tiling_on_sc=False),
    )


def _proj_body(x_ref, w_ref, b_ref, o_ref):
    o_ref[...] = (jnp.dot(x_ref[...], w_ref[...],
                          preferred_element_type=jnp.float32) + b_ref[...])


_proj = pl.pallas_call(
    _proj_body, grid=(N // PBN,),
    in_specs=[pl.BlockSpec((PBN, D), lambda i: (i, 0)),
              pl.BlockSpec((D, D), lambda i: (0, 0)),
              pl.BlockSpec((1, D), lambda i: (0, 0))],
    out_specs=pl.BlockSpec((PBN, D), lambda i: (i, 0)),
    out_shape=jax.ShapeDtypeStruct((N, D), jnp.float32),
)


def _combine_body(h_ref, a0_ref, a1_ref, inv_ref, rw_ref, w0_ref, w1_ref,
                  cb_ref, g_ref, b_ref, o_ref):
    h = h_ref[...]
    inv = inv_ref[0]
    out = (jnp.dot(h, rw_ref[...], preferred_element_type=jnp.float32)
           + cb_ref[...])
    sc = jnp.concatenate(
        [jnp.broadcast_to(inv[:, r:r + 1], (BN, HD)) for r in range(R)],
        axis=1)
    out = out + jnp.dot(a0_ref[...] * sc, w0_ref[...],
                        preferred_element_type=jnp.float32)
    out = out + jnp.dot(a1_ref[...] * sc, w1_ref[...],
                        preferred_element_type=jnp.float32)
    z = jnp.maximum(out + h, 0.0)
    mu = jnp.mean(z, axis=-1, keepdims=True)
    zc = z - mu
    var = jnp.mean(zc * zc, axis=-1, keepdims=True)
    o_ref[...] = zc * lax.rsqrt(var + 1e-5) * g_ref[...] + b_ref[...]


_combine = pl.pallas_call(
    _combine_body, grid=(NPAD // BN,),
    in_specs=[pl.BlockSpec((BN, D), lambda i: (i, 0)),
              pl.BlockSpec((BN, R * HD), lambda i: (i, 0)),
              pl.BlockSpec((BN, R * HD), lambda i: (i, 0)),
              pl.BlockSpec((1, BN, R), lambda i: (i, 0, 0)),
              pl.BlockSpec((D, D), lambda i: (0, 0)),
              pl.BlockSpec((R * HD, D), lambda i: (0, 0)),
              pl.BlockSpec((R * HD, D), lambda i: (0, 0)),
              pl.BlockSpec((1, D), lambda i: (0, 0)),
              pl.BlockSpec((1, D), lambda i: (0, 0)),
              pl.BlockSpec((1, D), lambda i: (0, 0))],
    out_specs=pl.BlockSpec((BN, D), lambda i: (i, 0)),
    out_shape=jax.ShapeDtypeStruct((N, D), jnp.float32),
)


def _pool_body(h_ref, b_ref, w1_ref, b1_ref, w2_ref, b2_ref, o_ref, hg):
    i = pl.program_id(0)

    @pl.when(i == 0)
    def _init():
        hg[...] = jnp.full((NGRAPH, D), -jnp.inf, jnp.float32)

    bb = b_ref[...]
    h = h_ref[...]
    for g in range(NGRAPH):
        m = jnp.max(jnp.where(bb == g, h, -jnp.inf), axis=0,
                    keepdims=True)
        hg[pl.ds(g, 1)] = jnp.maximum(hg[pl.ds(g, 1)], m)

    @pl.when(i == N // PBN - 1)
    def _head():
        hc = jnp.maximum(
            jnp.dot(hg[...], w1_ref[...], preferred_element_type=jnp.float32)
            + b1_ref[...], 0.0)
        o_ref[...] = (jnp.dot(hc, w2_ref[...],
                              preferred_element_type=jnp.float32)
                      + b2_ref[...])


_pool = pl.pallas_call(
    _pool_body, grid=(N // PBN,),
    in_specs=[pl.BlockSpec((PBN, D), lambda i: (i, 0)),
              pl.BlockSpec((PBN, 1), lambda i: (i, 0)),
              pl.BlockSpec((D, D), lambda i: (0, 0)),
              pl.BlockSpec((1, D), lambda i: (0, 0)),
              pl.BlockSpec((D, 4), lambda i: (0, 0)),
              pl.BlockSpec((1, 4), lambda i: (0, 0))],
    out_specs=pl.BlockSpec((NGRAPH, 4), lambda i: (0, 0)),
    out_shape=jax.ShapeDtypeStruct((NGRAPH, 4), jnp.float32),
    scratch_shapes=[pltpu.VMEM((NGRAPH, D), jnp.float32)],
)


def kernel(x, edge_index, edge_type, batch, params):
    src = edge_index[0]
    dst = edge_index[1]
    src_pad = jnp.concatenate([src, jnp.zeros((EPAD - E,), jnp.int32)])
    inv_flat, dst3 = _prep_kernel()(dst, edge_type)
    inv3 = inv_flat.reshape(NPAD // BN, BN, R)
    h = _proj(x, params['in_W'], params['in_b'].reshape(1, D))
    for i in range(3):
        a0, a1 = _agg_kernel()(src_pad, dst3, h.reshape(2 * N, HD))
        relw = params['rel_W'][i]
        w0 = relw[:, :HD, :].reshape(R * HD, D)
        w1 = relw[:, HD:, :].reshape(R * HD, D)
        h = _combine(h, a0.reshape(NPAD, R * HD), a1.reshape(NPAD, R * HD),
                     inv3, params['root_W'][i], w0, w1,
                     params['conv_b'][i].reshape(1, D),
                     params['ln_g'][i].reshape(1, D),
                     params['ln_b'][i].reshape(1, D))
    return _pool(h, batch.reshape(N, 1), params['cls_W1'],
                 params['cls_b1'].reshape(1, D), params['cls_W2'],
                 params['cls_b2'].reshape(1, 4))
